# baseline (speedup 1.0000x reference)
import functools

import jax
import jax.numpy as jnp
from jax import lax
from jax.experimental import pallas as pl
from jax.experimental.pallas import tpu as pltpu
from jax.experimental.pallas import tpu_sc as plsc

D_EMBED = 128
SEQ_LEN = 1000
BATCH = 16384

_info = plsc.get_sparse_core_info()
_NC = _info.num_cores
_NS = _info.num_subcores
_NW = _NC * _NS
_BPW = BATCH // _NW
_CHUNK = 128
_NCHUNK = _BPW // _CHUNK
_STAGERS = 16
_STAGE_ROWS = 64
_STAGE_LAST = SEQ_LEN - (_STAGERS - 1) * _STAGE_ROWS

_mesh = plsc.VectorSubcoreMesh(core_axis_name="c", subcore_axis_name="s")


@functools.partial(
    pl.kernel,
    mesh=_mesh,
    out_type=jax.ShapeDtypeStruct((BATCH, D_EMBED), jnp.float32),
    scratch_types=[
        pltpu.VMEM((_BPW,), jnp.int32),
        pltpu.VMEM((_BPW, D_EMBED), jnp.float32),
        pltpu.VMEM_SHARED((SEQ_LEN, D_EMBED), jnp.float32),
        pltpu.SemaphoreType.DMA,
        pltpu.SemaphoreType.DMA,
        pltpu.SemaphoreType.DMA,
        pltpu.SemaphoreType.DMA,
    ],
)
def _gather_kernel(pe_hbm, t_hbm, out_hbm, idx_v, rows_v, pe_sp,
                   gsem, wsem, ssem_a, ssem_b):
    sid = lax.axis_index("s")
    wid = sid * _NC + lax.axis_index("c")
    base = wid * _BPW
    pltpu.sync_copy(t_hbm.at[pl.ds(base, _BPW)], idx_v)

    g0 = pltpu.async_copy(
        pe_hbm.at[idx_v.at[pl.ds(0, _CHUNK)]],
        rows_v.at[pl.ds(0, _CHUNK)],
        gsem,
    )

    def _stage_rows(r0, n_a, n_b):
        ra = pltpu.async_copy(pe_hbm.at[pl.ds(r0, n_a)],
                              rows_v.at[pl.ds(_CHUNK, n_a)], ssem_a)
        rb = pltpu.async_copy(pe_hbm.at[pl.ds(r0 + n_a, n_b)],
                              rows_v.at[pl.ds(_CHUNK + n_a, n_b)], ssem_b)
        ra.wait()
        wa = pltpu.async_copy(rows_v.at[pl.ds(_CHUNK, n_a)],
                              pe_sp.at[pl.ds(r0, n_a)], ssem_a)
        rb.wait()
        wb = pltpu.async_copy(rows_v.at[pl.ds(_CHUNK + n_a, n_b)],
                              pe_sp.at[pl.ds(r0 + n_a, n_b)], ssem_b)
        wa.wait()
        wb.wait()

    @pl.when(sid < _STAGERS - 1)
    def _stage():
        _stage_rows(sid * _STAGE_ROWS, _STAGE_ROWS // 2, _STAGE_ROWS // 2)

    @pl.when(sid == _STAGERS - 1)
    def _stage_last():
        _stage_rows((_STAGERS - 1) * _STAGE_ROWS, 32, _STAGE_LAST - 32)

    plsc.subcore_barrier()

    gathers = [g0] + [
        pltpu.async_copy(
            pe_sp.at[idx_v.at[pl.ds(j * _CHUNK, _CHUNK)]],
            rows_v.at[pl.ds(j * _CHUNK, _CHUNK)],
            gsem,
        )
        for j in range(1, _NCHUNK)
    ]
    writes = []
    for j in range(_NCHUNK):
        gathers[j].wait()
        writes.append(
            pltpu.async_copy(
                rows_v.at[pl.ds(j * _CHUNK, _CHUNK)],
                out_hbm.at[pl.ds(base + j * _CHUNK, _CHUNK)],
                wsem,
            )
        )
    for w in writes:
        w.wait()


def kernel(pe, t):
    return _gather_kernel(pe, t.astype(jnp.int32))


# speedup vs baseline: 1.0160x; 1.0160x over previous
import functools

import jax
import jax.numpy as jnp
from jax import lax
from jax.experimental import pallas as pl
from jax.experimental.pallas import tpu as pltpu
from jax.experimental.pallas import tpu_sc as plsc

D_EMBED = 128
SEQ_LEN = 1000
BATCH = 16384

_info = plsc.get_sparse_core_info()
_NC = _info.num_cores
_NS = _info.num_subcores
_NW = _NC * _NS
_BPW = BATCH // _NW
_CHUNK = 128
_NCHUNK = _BPW // _CHUNK
_STAGERS = 16
_STAGE_ROWS = 64
_STAGE_LAST = SEQ_LEN - (_STAGERS - 1) * _STAGE_ROWS

_mesh = plsc.VectorSubcoreMesh(core_axis_name="c", subcore_axis_name="s")


@functools.partial(
    pl.kernel,
    mesh=_mesh,
    out_type=jax.ShapeDtypeStruct((BATCH, D_EMBED), jnp.float32),
    scratch_types=[
        pltpu.VMEM((_BPW,), jnp.int32),
        pltpu.VMEM((_BPW, D_EMBED), jnp.float32),
        pltpu.VMEM_SHARED((SEQ_LEN, D_EMBED), jnp.float32),
        pltpu.SemaphoreType.DMA,
        pltpu.SemaphoreType.DMA,
    ],
)
def _gather_kernel(pe_hbm, t_hbm, out_hbm, idx_v, rows_v, pe_sp, gsem, wsem):
    sid = lax.axis_index("s")
    wid = sid * _NC + lax.axis_index("c")
    base = wid * _BPW
    pltpu.sync_copy(t_hbm.at[pl.ds(base, _BPW)], idx_v)

    g0 = pltpu.async_copy(
        pe_hbm.at[idx_v.at[pl.ds(0, _CHUNK)]],
        rows_v.at[pl.ds(0, _CHUNK)],
        gsem,
    )

    @pl.when(sid < _STAGERS - 1)
    def _stage():
        r0 = sid * _STAGE_ROWS
        pltpu.sync_copy(pe_hbm.at[pl.ds(r0, _STAGE_ROWS)],
                        rows_v.at[pl.ds(_CHUNK, _STAGE_ROWS)])
        pltpu.sync_copy(rows_v.at[pl.ds(_CHUNK, _STAGE_ROWS)],
                        pe_sp.at[pl.ds(r0, _STAGE_ROWS)])

    @pl.when(sid == _STAGERS - 1)
    def _stage_last():
        r0 = (_STAGERS - 1) * _STAGE_ROWS
        pltpu.sync_copy(pe_hbm.at[pl.ds(r0, _STAGE_LAST)],
                        rows_v.at[pl.ds(_CHUNK, _STAGE_LAST)])
        pltpu.sync_copy(rows_v.at[pl.ds(_CHUNK, _STAGE_LAST)],
                        pe_sp.at[pl.ds(r0, _STAGE_LAST)])

    plsc.subcore_barrier()

    gathers = [g0] + [
        pltpu.async_copy(
            pe_sp.at[idx_v.at[pl.ds(j * _CHUNK, _CHUNK)]],
            rows_v.at[pl.ds(j * _CHUNK, _CHUNK)],
            gsem,
        )
        for j in range(1, _NCHUNK)
    ]
    writes = []
    for j in range(_NCHUNK):
        gathers[j].wait()
        writes.append(
            pltpu.async_copy(
                rows_v.at[pl.ds(j * _CHUNK, _CHUNK)],
                out_hbm.at[pl.ds(base + j * _CHUNK, _CHUNK)],
                wsem,
            )
        )
    for w in writes:
        w.wait()


def kernel(pe, t):
    return _gather_kernel(pe, t.astype(jnp.int32))
